# Initial kernel scaffold; baseline (speedup 1.0000x reference)
import jax
import jax.numpy as jnp
from jax.experimental import pallas as pl


def kernel(t, c, context_table, target_table):
    raise NotImplementedError("write your pallas kernel here")


# speedup vs baseline: 1.6120x; 1.6120x over previous
import functools

import jax
import jax.numpy as jnp
from jax import lax
from jax.experimental import pallas as pl
from jax.experimental.pallas import tpu as pltpu
from jax.experimental.pallas import tpu_sc as plsc

_VOCAB = 100000
_D = 128
_NEG = 50
_NEG_PAD = 64
_B = 16384

_NW = 32
_BPW = _B // _NW
_CB = 16
_NCHUNK = _BPW // _CB
_ROWS = _CB * _NEG
_GCH = [(j * 128, 128) for j in range(6)] + [(768, 32)]


_GATHER_DNUMS = lax.GatherDimensionNumbers(
    offset_dims=(), collapsed_slice_dims=(0,), start_index_map=(0,))


def _lane_bcast(vec, lane_idx):
    idx = jnp.full((16, 1), lane_idx, jnp.int32)
    return lax.gather(vec, idx, _GATHER_DNUMS, slice_sizes=(1,),
                      mode=lax.GatherScatterMode.PROMISE_IN_BOUNDS)


def _body(t_ref, c_ref, ctab, ttab, out_ref,
          t_idx, c_idx, ctx_v, tgt_v, z_v, sem):
    wid = lax.axis_index("s") * 2 + lax.axis_index("c")
    lane = lax.iota(jnp.int32, 16)

    def chunk_body(ch, _):
        base = wid * _BPW + ch * _CB

        pltpu.sync_copy(t_ref.at[pl.ds(base, _CB)], t_idx)
        pltpu.sync_copy(c_ref.at[pl.ds(base * _NEG, _ROWS)], c_idx)

        copies = [pltpu.async_copy(ctab.at[t_idx], ctx_v, sem)]
        for off, sz in _GCH:
            copies.append(pltpu.async_copy(
                ttab.at[c_idx.at[pl.ds(off, sz)]],
                tgt_v.at[pl.ds(off, sz)], sem))
        for cp in copies:
            cp.wait()

        def b_body(b, _):
            def g_body(g, _):
                n = g * 16 + lane
                valid = n < _NEG
                rows = b * _NEG + jnp.where(valid, n, 0)
                acc = jnp.zeros((16,), jnp.float32)
                for k in range(_D // 16):
                    ctx_k = ctx_v[b, pl.ds(k * 16, 16)]
                    for dd in range(16):
                        d = k * 16 + dd
                        bc = _lane_bcast(ctx_k, dd)
                        col = plsc.load_gather(
                            tgt_v, [rows, jnp.full((16,), d, jnp.int32)])
                        acc = acc + bc * col
                z_v[b, pl.ds(g * 16, 16)] = jnp.where(
                    valid, acc, jnp.float32(-jnp.inf))
                return 0

            lax.fori_loop(0, _NEG_PAD // 16, g_body, 0)

            zs = [z_v[b, pl.ds(j * 16, 16)] for j in range(_NEG_PAD // 16)]
            m = jnp.max(jnp.maximum(jnp.maximum(zs[0], zs[1]),
                                    jnp.maximum(zs[2], zs[3])))
            es = [jnp.exp(zj - m) for zj in zs]
            s = jnp.sum(es[0] + es[1] + es[2] + es[3])
            for j in range(_NEG_PAD // 16):
                z_v[b, pl.ds(j * 16, 16)] = es[j] / s
            return 0

        lax.fori_loop(0, _CB, b_body, 0)
        pltpu.sync_copy(z_v, out_ref.at[pl.ds(base, _CB)])
        return 0

    lax.fori_loop(0, _NCHUNK, chunk_body, 0)


@jax.jit
def kernel(t, c, context_table, target_table):
    t_flat = t.reshape(_B)
    c_flat = c.reshape(_B * _NEG)
    k = functools.partial(
        pl.kernel,
        out_type=jax.ShapeDtypeStruct((_B, _NEG_PAD), jnp.float32),
        mesh=plsc.VectorSubcoreMesh(core_axis_name="c", subcore_axis_name="s"),
        compiler_params=pltpu.CompilerParams(needs_layout_passes=False),
        scratch_types=[
            pltpu.VMEM((_CB,), jnp.int32),
            pltpu.VMEM((_ROWS,), jnp.int32),
            pltpu.VMEM((_CB, _D), jnp.float32),
            pltpu.VMEM((_ROWS, _D), jnp.float32),
            pltpu.VMEM((_CB, _NEG_PAD), jnp.float32),
            pltpu.SemaphoreType.DMA,
        ],
    )(_body)
    out = k(t_flat, c_flat, context_table, target_table)
    return out[:, :_NEG]
